# Initial kernel scaffold; baseline (speedup 1.0000x reference)
import jax
import jax.numpy as jnp
from jax.experimental import pallas as pl


def kernel(h_s, h_v, edge_index, pos, orientation, W1, b1, W2, b2):
    raise NotImplementedError("write your pallas kernel here")


# speedup vs baseline: 1.8799x; 1.8799x over previous
import functools

import jax
import jax.numpy as jnp
from jax.experimental import pallas as pl
from jax.experimental.pallas import tpu as pltpu

N_NODES = 50000
N_EDGES = 800000
SD = 64
VD = 64
HID = 128

E_B = 2000
N_B = 10000


def _prep_body(pos_ref, ori_ref, t_ref):
    a2 = 2.0 * ori_ref[:, 0:1]
    t_ref[:, 0:2] = pos_ref[:, :]
    t_ref[:, 2:3] = jnp.cos(a2)
    t_ref[:, 3:4] = jnp.sin(a2)


def _node_table(pos, orientation):
    return pl.pallas_call(
        _prep_body,
        grid=(N_NODES // N_B,),
        in_specs=[
            pl.BlockSpec((N_B, 2), lambda i: (i, 0)),
            pl.BlockSpec((N_B, 1), lambda i: (i, 0)),
        ],
        out_specs=pl.BlockSpec((N_B, 4), lambda i: (i, 0)),
        out_shape=jax.ShapeDtypeStruct((N_NODES, 4), jnp.float32),
    )(pos, orientation)


def _mlp_body(hs_s_ref, hs_d_ref, hv_s_ref, t8_ref,
              w1a_ref, w1b_ref, w1c_ref, w1cp_ref, wgeo_ref, b1_ref,
              w2t_ref, b2_ref, msg_ref):
    t8 = t8_ref[:, :]
    dx = t8[:, 0:1] - t8[:, 4:5]
    dy = t8[:, 1:2] - t8[:, 5:6]
    r2 = dx * dx + dy * dy
    nz = r2 > 0.0
    inv = jnp.where(nz, 1.0 / jnp.where(nz, r2, 1.0), 0.0)
    dist = jnp.sqrt(r2) + 1e-6
    c2p = jnp.where(nz, (dx * dx - dy * dy) * inv, 1.0)
    s2p = 2.0 * dx * dy * inv
    c2d = t8[:, 6:7]
    s2d = t8[:, 7:8]
    c2s = t8[:, 2:3]
    s2s = t8[:, 3:4]
    cg = c2p * c2d + s2p * s2d
    sg = s2p * c2d - c2p * s2d
    cr = c2s * c2d + s2s * s2d
    sr = s2s * c2d - c2s * s2d
    hv = hv_s_ref[:, :]
    hid = jnp.dot(hs_s_ref[:, :], w1a_ref[:, :], preferred_element_type=jnp.float32)
    hid += jnp.dot(hs_d_ref[:, :], w1b_ref[:, :], preferred_element_type=jnp.float32)
    hid += cr * jnp.dot(hv, w1c_ref[:, :], preferred_element_type=jnp.float32)
    hid += sr * jnp.dot(hv, w1cp_ref[:, :], preferred_element_type=jnp.float32)
    hid += dist * wgeo_ref[0:1, :] + cg * wgeo_ref[1:2, :] + sg * wgeo_ref[2:3, :]
    hid += b1_ref[:, :]
    hid = hid * (1.0 / (1.0 + jnp.exp(-hid)))
    msg_ref[:, :] = (jnp.dot(hid, w2t_ref[:, :], preferred_element_type=jnp.float32)
                     + b2_ref[:, :])


def _edge_mlp(hs_s, hs_d, hv_s, t8, w1a, w1b, w1c, w1cp, wgeo, b1, w2t, b2):
    ew = lambda c: pl.BlockSpec((E_B, c), lambda i: (i, 0))
    full = lambda r, c: pl.BlockSpec((r, c), lambda i: (0, 0))
    return pl.pallas_call(
        _mlp_body,
        grid=(N_EDGES // E_B,),
        in_specs=[
            ew(SD), ew(SD), ew(VD), ew(8),
            full(SD, HID), full(SD, HID), full(VD, HID), full(VD, HID),
            full(8, HID), full(1, HID), full(HID, HID), full(1, HID),
        ],
        out_specs=ew(HID),
        out_shape=jax.ShapeDtypeStruct((N_EDGES, HID), jnp.float32),
    )(hs_s, hs_d, hv_s, t8, w1a, w1b, w1c, w1cp, wgeo, b1, w2t, b2)


def kernel(h_s, h_v, edge_index, pos, orientation, W1, b1, W2, b2):
    src = edge_index[0]
    dst = edge_index[1]
    w1a = W1[:, :SD].T
    w1b = W1[:, SD:2 * SD].T
    w1c = W1[:, 2 * SD:2 * SD + VD].T
    w1cp = jnp.zeros_like(w1c)
    w1cp = w1cp.at[0::2, :].set(w1c[1::2, :]).at[1::2, :].set(-w1c[0::2, :])
    wgeo = jnp.concatenate([W1[:, 2 * SD + VD:].T,
                            jnp.zeros((5, HID), jnp.float32)], axis=0)
    b1r = b1.reshape(1, HID)
    w2t = W2.T
    b2r = b2.reshape(1, HID)

    T = _node_table(pos, orientation)

    hs_s = h_s[src]
    hs_d = h_s[dst]
    hv_s = h_v[src]
    t8 = jnp.concatenate([T[src], T[dst]], axis=1)

    msg = _edge_mlp(hs_s, hs_d, hv_s, t8,
                    w1a, w1b, w1c, w1cp, wgeo, b1r, w2t, b2r)

    out_s = h_s.at[dst].add(msg[:, :SD])
    out_v = h_v.at[dst].add(msg[:, SD:])
    return (out_s, out_v)


# speedup vs baseline: 4.1484x; 2.2068x over previous
import jax
import jax.numpy as jnp
from jax import lax
from jax.experimental import pallas as pl
from jax.experimental.pallas import tpu as pltpu
from jax.experimental.pallas import tpu_sc as plsc

N_NODES = 50000
N_EDGES = 800000
SD = 64
VD = 64
HID = 128

NC = 2
NS = 16
NW = NC * NS

E_B = 2000
N_B = 10000


def _mesh():
    return plsc.VectorSubcoreMesh(core_axis_name="c", subcore_axis_name="s",
                                  num_cores=NC, num_subcores=NS)


def _prep_body(pos_ref, ori_ref, t_ref):
    a2 = 2.0 * ori_ref[:, 0:1]
    t_ref[:, 0:2] = pos_ref[:, :]
    t_ref[:, 2:3] = jnp.cos(a2)
    t_ref[:, 3:4] = jnp.sin(a2)


def _node_table(pos, orientation):
    return pl.pallas_call(
        _prep_body,
        grid=(N_NODES // N_B,),
        in_specs=[
            pl.BlockSpec((N_B, 2), lambda i: (i, 0)),
            pl.BlockSpec((N_B, 1), lambda i: (i, 0)),
        ],
        out_specs=pl.BlockSpec((N_B, 4), lambda i: (i, 0)),
        out_shape=jax.ShapeDtypeStruct((N_NODES, 4), jnp.float32),
    )(pos, orientation)


GU = 256
N_UNITS = N_EDGES // GU
UPW = (N_UNITS + NW - 1) // NW


def _gather_body(src_a, dst_a, ta, tb, o_x1, o_x2, sids, dids, b1, b2, sem):
    wid = lax.axis_index("s") * NC + lax.axis_index("c")

    def unit(i, carry):
        u = wid + i * NW

        @pl.when(u < N_UNITS)
        def _():
            eb = u * GU
            pltpu.sync_copy(src_a.at[pl.ds(eb, GU)], sids)
            pltpu.sync_copy(dst_a.at[pl.ds(eb, GU)], dids)
            cps = []
            for j in range(GU // 128):
                sl = pl.ds(j * 128, 128)
                cps.append(pltpu.async_copy(ta.at[sids.at[sl]], b1.at[sl], sem))
                cps.append(pltpu.async_copy(tb.at[dids.at[sl]], b2.at[sl], sem))
            for c in cps:
                c.wait()
            pltpu.sync_copy(b1, o_x1.at[pl.ds(eb, GU)])
            pltpu.sync_copy(b2, o_x2.at[pl.ds(eb, GU)])
        return carry

    lax.fori_loop(0, UPW, unit, None)


def _sc_gather(src_a, dst_a, ta, tb):
    f32 = jnp.float32
    out_type = [
        jax.ShapeDtypeStruct((N_EDGES, HID), f32),
        jax.ShapeDtypeStruct((N_EDGES, HID), f32),
    ]
    scratch = [
        pltpu.VMEM((GU,), jnp.int32),
        pltpu.VMEM((GU,), jnp.int32),
        pltpu.VMEM((GU, HID), f32),
        pltpu.VMEM((GU, HID), f32),
        pltpu.SemaphoreType.DMA,
    ]
    return pl.kernel(_gather_body, out_type=out_type, mesh=_mesh(),
                     compiler_params=pltpu.CompilerParams(needs_layout_passes=False),
                     scratch_types=scratch)(src_a, dst_a, ta, tb)


def _mlp_body(x1_ref, x2_ref,
              w1a_ref, w1b_ref, we_ref, wo_ref, wgeo_ref, b1_ref,
              w2t_ref, b2_ref, msg_ref):
    x1 = x1_ref[:, :]
    x2 = x2_ref[:, :]
    sx = x1[:, 96:97]
    sy = x1[:, 97:98]
    c2s = x1[:, 98:99]
    s2s = x1[:, 99:100]
    dxp = x2[:, 64:65]
    dyp = x2[:, 65:66]
    c2d = x2[:, 66:67]
    s2d = x2[:, 67:68]
    dx = sx - dxp
    dy = sy - dyp
    r2 = dx * dx + dy * dy
    nz = r2 > 0.0
    inv = jnp.where(nz, 1.0 / jnp.where(nz, r2, 1.0), 0.0)
    dist = jnp.sqrt(r2) + 1e-6
    c2p = jnp.where(nz, (dx * dx - dy * dy) * inv, 1.0)
    s2p = 2.0 * dx * dy * inv
    cg = c2p * c2d + s2p * s2d
    sg = s2p * c2d - c2p * s2d
    cr = c2s * c2d + s2s * s2d
    sr = s2s * c2d - c2s * s2d
    hvp = lax.bitcast_convert_type(x1[:, 64:96], jnp.uint32)
    hv_e = lax.bitcast_convert_type(lax.shift_left(hvp, jnp.uint32(16)), jnp.float32)
    hv_o = lax.bitcast_convert_type(
        lax.bitwise_and(hvp, jnp.uint32(0xFFFF0000)), jnp.float32)
    vrx = cr * hv_e - sr * hv_o
    vry = sr * hv_e + cr * hv_o
    hid = jnp.dot(x1[:, 0:SD], w1a_ref[:, :], preferred_element_type=jnp.float32)
    hid += jnp.dot(x2[:, 0:SD], w1b_ref[:, :], preferred_element_type=jnp.float32)
    hid += jnp.dot(vrx, we_ref[:, :], preferred_element_type=jnp.float32)
    hid += jnp.dot(vry, wo_ref[:, :], preferred_element_type=jnp.float32)
    hid += dist * wgeo_ref[0:1, :] + cg * wgeo_ref[1:2, :] + sg * wgeo_ref[2:3, :]
    hid += b1_ref[:, :]
    hid = hid * (1.0 / (1.0 + jnp.exp(-hid)))
    msg_ref[:, :] = (jnp.dot(hid, w2t_ref[:, :], preferred_element_type=jnp.float32)
                     + b2_ref[:, :])


def _edge_mlp(x1, x2, w1a, w1b, we, wo, wgeo, b1, w2t, b2):
    ew = lambda c: pl.BlockSpec((E_B, c), lambda i: (i, 0))
    full = lambda r, c: pl.BlockSpec((r, c), lambda i: (0, 0))
    return pl.pallas_call(
        _mlp_body,
        grid=(N_EDGES // E_B,),
        in_specs=[
            ew(HID), ew(HID),
            full(SD, HID), full(SD, HID), full(VD // 2, HID), full(VD // 2, HID),
            full(8, HID), full(1, HID), full(HID, HID), full(1, HID),
        ],
        out_specs=ew(HID),
        out_shape=jax.ShapeDtypeStruct((N_EDGES, HID), jnp.float32),
    )(x1, x2, w1a, w1b, we, wo, wgeo, b1, w2t, b2)


NPAD = 50016
CHUNK = 12504
N_CHUNKS = 2
TRASH = CHUNK
EPT = N_EDGES // NS
BD = 2000
FB = 2048


def _scatter_body(dst_a, msg, hsv, out_sv, dids, locb, eidb, rows, acc, sem):
    cid = lax.axis_index("c")
    sid = lax.axis_index("s")
    e0 = sid * EPT
    trash_vec = jnp.full((16,), TRASH, jnp.int32)
    zero_vec = jnp.zeros((16,), jnp.int32)
    lane = lax.iota(jnp.int32, 16)

    def run_chunk(k, acc):
        lo = cid * (CHUNK * N_CHUNKS) + k * CHUNK

        @pl.when(sid == 0)
        def _():
            pltpu.sync_copy(hsv.at[pl.ds(lo, CHUNK)], acc.at[pl.ds(0, CHUNK)])

        plsc.subcore_barrier()

        def batch(b, carry):
            pltpu.sync_copy(dst_a.at[pl.ds(e0 + b * BD, BD)], dids)
            for r in range(FB // 128):
                for l in range(8):
                    locb[r, pl.ds(l * 16, 16)] = trash_vec
                    eidb[r, pl.ds(l * 16, 16)] = zero_vec

            def group(g, cnt):
                ids16 = dids[pl.ds(g * 16, 16)]
                rel = ids16 - lo
                mask = (rel >= 0) & (rel < CHUNK)
                csum = jnp.where(mask, 1, 0)
                for d in (1, 2, 4, 8):
                    sh = csum.at[jnp.maximum(lane - d, 0)].get(
                        mode="promise_in_bounds")
                    csum = csum + jnp.where(lane >= d, sh, 0)
                pos = cnt + csum - 1
                row = lax.shift_right_logical(pos, 7)
                col = lax.bitwise_and(pos, 127)
                plsc.store_scatter(locb, [row, col], rel, mask=mask)
                eidv = jnp.full((16,), e0 + b * BD + g * 16, jnp.int32) + lane
                plsc.store_scatter(eidb, [row, col], eidv, mask=mask)
                return cnt + csum[15]

            cnt = lax.fori_loop(0, BD // 16, group, zero_vec)
            nflush = lax.shift_right_logical(cnt[0] + 127, 7)

            def flush(f, c2):
                pltpu.async_copy(msg.at[eidb.at[f]], rows, sem).wait()
                pltpu.sync_copy(rows, acc.at[locb.at[f]], add=True)
                return c2

            lax.fori_loop(0, nflush, flush, None)
            return carry

        lax.fori_loop(0, EPT // BD, batch, None)
        plsc.subcore_barrier()

        @pl.when(sid == 0)
        def _():
            pltpu.sync_copy(acc.at[pl.ds(0, CHUNK)], out_sv.at[pl.ds(lo, CHUNK)])

        plsc.subcore_barrier()

    def step(k, carry):
        run_chunk(k, acc)
        return carry

    lax.fori_loop(0, N_CHUNKS, step, None)


def _sc_scatter(dst_a, msg, hsv):
    f32 = jnp.float32
    out_type = jax.ShapeDtypeStruct((NPAD, HID), f32)
    scratch = [
        pltpu.VMEM((BD,), jnp.int32),
        pltpu.VMEM((FB // 128, 128), jnp.int32),
        pltpu.VMEM((FB // 128, 128), jnp.int32),
        pltpu.VMEM((128, HID), f32),
        pltpu.VMEM_SHARED((CHUNK + 8, HID), f32),
        pltpu.SemaphoreType.DMA,
    ]
    return pl.kernel(_scatter_body, out_type=out_type, mesh=_mesh(),
                     compiler_params=pltpu.CompilerParams(needs_layout_passes=False),
                     scratch_types=scratch)(dst_a, msg, hsv)


def kernel(h_s, h_v, edge_index, pos, orientation, W1, b1, W2, b2):
    w1a = W1[:, :SD].T
    w1b = W1[:, SD:2 * SD].T
    w1c = W1[:, 2 * SD:2 * SD + VD].T
    we = w1c[0::2, :]
    wo = w1c[1::2, :]
    wgeo = jnp.concatenate([W1[:, 2 * SD + VD:].T,
                            jnp.zeros((5, HID), jnp.float32)], axis=0)
    b1r = b1.reshape(1, HID)
    w2t = W2.T
    b2r = b2.reshape(1, HID)

    T = _node_table(pos, orientation)

    hvb = h_v.astype(jnp.bfloat16)
    lo16 = lax.bitcast_convert_type(hvb[:, 0::2], jnp.uint16).astype(jnp.uint32)
    hi16 = lax.bitcast_convert_type(hvb[:, 1::2], jnp.uint16).astype(jnp.uint32)
    hv_packed = lax.bitcast_convert_type(lo16 | (hi16 << jnp.uint32(16)),
                                         jnp.float32)
    padA = jnp.zeros((N_NODES, HID - SD - VD // 2 - 4), jnp.float32)
    ta = jnp.concatenate([h_s, hv_packed, T, padA], axis=1)
    padB = jnp.zeros((N_NODES, HID - SD - 4), jnp.float32)
    tb = jnp.concatenate([h_s, T, padB], axis=1)

    src_a = edge_index[0]
    dst_a = edge_index[1]
    x1, x2 = _sc_gather(src_a, dst_a, ta, tb)
    msg = _edge_mlp(x1, x2, w1a, w1b, we, wo, wgeo, b1r, w2t, b2r)
    hsv = jnp.concatenate(
        [jnp.concatenate([h_s, h_v], axis=1),
         jnp.zeros((NPAD - N_NODES, HID), jnp.float32)], axis=0)
    out_sv = _sc_scatter(dst_a, msg, hsv)
    return (out_sv[:N_NODES, :SD], out_sv[:N_NODES, SD:])


# speedup vs baseline: 5.0974x; 1.2288x over previous
import jax
import jax.numpy as jnp
from jax import lax
from jax.experimental import pallas as pl
from jax.experimental.pallas import tpu as pltpu
from jax.experimental.pallas import tpu_sc as plsc

N_NODES = 50000
N_EDGES = 800000
SD = 64
VD = 64
HID = 128

NC = 2
NS = 16
NW = NC * NS

E_B = 2000
N_B = 10000


def _mesh():
    return plsc.VectorSubcoreMesh(core_axis_name="c", subcore_axis_name="s",
                                  num_cores=NC, num_subcores=NS)


def _prep_body(pos_ref, ori_ref, t_ref):
    a2 = 2.0 * ori_ref[:, 0:1]
    t_ref[:, 0:2] = pos_ref[:, :]
    t_ref[:, 2:3] = jnp.cos(a2)
    t_ref[:, 3:4] = jnp.sin(a2)


def _node_table(pos, orientation):
    return pl.pallas_call(
        _prep_body,
        grid=(N_NODES // N_B,),
        in_specs=[
            pl.BlockSpec((N_B, 2), lambda i: (i, 0)),
            pl.BlockSpec((N_B, 1), lambda i: (i, 0)),
        ],
        out_specs=pl.BlockSpec((N_B, 4), lambda i: (i, 0)),
        out_shape=jax.ShapeDtypeStruct((N_NODES, 4), jnp.float32),
    )(pos, orientation)


GU = 128
N_UNITS = N_EDGES // GU
UPW = (N_UNITS + NW - 1) // NW
NBUF = 3
J_TOT = UPW + 2


def _gather_body(src_a, dst_a, ta, tb, o_x1, o_x2, sids, dids, b1, b2,
                 si0, si1, si2, sg0, sg1, sg2, ss0, ss1, ss2):
    wid = lax.axis_index("s") * NC + lax.axis_index("c")
    sem_i = (si0, si1, si2)
    sem_g = (sg0, sg1, sg2)
    sem_s = (ss0, ss1, ss2)

    def valid(k):
        return (k >= 0) & (k < UPW) & (wid + k * NW < N_UNITS)

    def fire_ids(k, p):
        eb = (wid + k * NW) * GU
        pltpu.async_copy(src_a.at[pl.ds(eb, GU)], sids.at[p], sem_i[p])
        pltpu.async_copy(dst_a.at[pl.ds(eb, GU)], dids.at[p], sem_i[p])

    def wait_ids(p):
        pltpu.make_async_copy(src_a.at[pl.ds(0, GU)], sids.at[p], sem_i[p]).wait()
        pltpu.make_async_copy(dst_a.at[pl.ds(0, GU)], dids.at[p], sem_i[p]).wait()

    def fire_gathers(p):
        pltpu.async_copy(ta.at[sids.at[p]], b1.at[p], sem_g[p])
        pltpu.async_copy(tb.at[dids.at[p]], b2.at[p], sem_g[p])

    def wait_gathers(p):
        pltpu.make_async_copy(ta.at[sids.at[p]], b1.at[p], sem_g[p]).wait()
        pltpu.make_async_copy(tb.at[dids.at[p]], b2.at[p], sem_g[p]).wait()

    def fire_stores(k, p):
        eb = (wid + k * NW) * GU
        pltpu.async_copy(b1.at[p], o_x1.at[pl.ds(eb, GU)], ss0 if p == 0 else (ss1 if p == 1 else ss2))
        pltpu.async_copy(b2.at[p], o_x2.at[pl.ds(eb, GU)], sem_s[p])

    def wait_stores(p):
        pltpu.make_async_copy(b1.at[p], o_x1.at[pl.ds(0, GU)], sem_s[p]).wait()
        pltpu.make_async_copy(b2.at[p], o_x2.at[pl.ds(0, GU)], sem_s[p]).wait()

    def tri(it, carry):
        for o in range(3):
            j = it * 3 + o
            @pl.when(valid(j))
            def _():
                fire_ids(j, o)

            p1 = (o - 1) % 3

            @pl.when(valid(j - 4))
            def _():
                wait_stores(p1)

            @pl.when(valid(j - 1))
            def _():
                wait_ids(p1)
                fire_gathers(p1)

            p2 = (o - 2) % 3

            @pl.when(valid(j - 2))
            def _():
                wait_gathers(p2)
                fire_stores(j - 2, p2)
        return carry

    lax.fori_loop(0, J_TOT // 3, tri, None)
    for k_off in range(2):
        k = UPW - 2 + k_off
        p = k % 3

        @pl.when(valid(k))
        def _():
            wait_stores(p)


def _sc_gather(src_a, dst_a, ta, tb):
    f32 = jnp.float32
    out_type = [
        jax.ShapeDtypeStruct((N_EDGES, HID), f32),
        jax.ShapeDtypeStruct((N_EDGES, HID), f32),
    ]
    scratch = [
        pltpu.VMEM((NBUF, GU), jnp.int32),
        pltpu.VMEM((NBUF, GU), jnp.int32),
        pltpu.VMEM((NBUF, GU, HID), f32),
        pltpu.VMEM((NBUF, GU, HID), f32),
    ] + [pltpu.SemaphoreType.DMA] * 9
    return pl.kernel(_gather_body, out_type=out_type, mesh=_mesh(),
                     compiler_params=pltpu.CompilerParams(needs_layout_passes=False),
                     scratch_types=scratch)(src_a, dst_a, ta, tb)


def _mlp_body(x1_ref, x2_ref,
              w1a_ref, w1b_ref, we_ref, wo_ref, wgeo_ref, b1_ref,
              w2t_ref, b2_ref, msg_ref):
    x1 = x1_ref[:, :]
    x2 = x2_ref[:, :]
    sx = x1[:, 96:97]
    sy = x1[:, 97:98]
    c2s = x1[:, 98:99]
    s2s = x1[:, 99:100]
    dxp = x2[:, 64:65]
    dyp = x2[:, 65:66]
    c2d = x2[:, 66:67]
    s2d = x2[:, 67:68]
    dx = sx - dxp
    dy = sy - dyp
    r2 = dx * dx + dy * dy
    nz = r2 > 0.0
    inv = jnp.where(nz, 1.0 / jnp.where(nz, r2, 1.0), 0.0)
    dist = jnp.sqrt(r2) + 1e-6
    c2p = jnp.where(nz, (dx * dx - dy * dy) * inv, 1.0)
    s2p = 2.0 * dx * dy * inv
    cg = c2p * c2d + s2p * s2d
    sg = s2p * c2d - c2p * s2d
    cr = c2s * c2d + s2s * s2d
    sr = s2s * c2d - c2s * s2d
    hvp = lax.bitcast_convert_type(x1[:, 64:96], jnp.uint32)
    hv_e = lax.bitcast_convert_type(lax.shift_left(hvp, jnp.uint32(16)), jnp.float32)
    hv_o = lax.bitcast_convert_type(
        lax.bitwise_and(hvp, jnp.uint32(0xFFFF0000)), jnp.float32)
    vrx = cr * hv_e - sr * hv_o
    vry = sr * hv_e + cr * hv_o
    hid = jnp.dot(x1[:, 0:SD], w1a_ref[:, :], preferred_element_type=jnp.float32)
    hid += jnp.dot(x2[:, 0:SD], w1b_ref[:, :], preferred_element_type=jnp.float32)
    hid += jnp.dot(vrx, we_ref[:, :], preferred_element_type=jnp.float32)
    hid += jnp.dot(vry, wo_ref[:, :], preferred_element_type=jnp.float32)
    hid += dist * wgeo_ref[0:1, :] + cg * wgeo_ref[1:2, :] + sg * wgeo_ref[2:3, :]
    hid += b1_ref[:, :]
    hid = hid * (1.0 / (1.0 + jnp.exp(-hid)))
    msg_ref[:, :] = (jnp.dot(hid, w2t_ref[:, :], preferred_element_type=jnp.float32)
                     + b2_ref[:, :])


def _edge_mlp(x1, x2, w1a, w1b, we, wo, wgeo, b1, w2t, b2):
    ew = lambda c: pl.BlockSpec((E_B, c), lambda i: (i, 0))
    full = lambda r, c: pl.BlockSpec((r, c), lambda i: (0, 0))
    return pl.pallas_call(
        _mlp_body,
        grid=(N_EDGES // E_B,),
        in_specs=[
            ew(HID), ew(HID),
            full(SD, HID), full(SD, HID), full(VD // 2, HID), full(VD // 2, HID),
            full(8, HID), full(1, HID), full(HID, HID), full(1, HID),
        ],
        out_specs=ew(HID),
        out_shape=jax.ShapeDtypeStruct((N_EDGES, HID), jnp.float32),
    )(x1, x2, w1a, w1b, we, wo, wgeo, b1, w2t, b2)


NPAD = 50016
CHUNK = 12504
N_CHUNKS = 2
TRASH = CHUNK
EPT = N_EDGES // NS
BD = 2000
FB = 2048
FL = 64


def _scatter_body(dst_a, msg, hsv, out_sv, dids, locb, eidb, rows0, rows1,
                  acc, sem, sg0, sg1, sa0, sa1):
    cid = lax.axis_index("c")
    sid = lax.axis_index("s")
    e0 = sid * EPT
    trash_vec = jnp.full((16,), TRASH, jnp.int32)
    zero_vec = jnp.zeros((16,), jnp.int32)
    lane = lax.iota(jnp.int32, 16)

    def run_chunk(k, acc):
        lo = cid * (CHUNK * N_CHUNKS) + k * CHUNK

        @pl.when(sid == 0)
        def _():
            pltpu.sync_copy(hsv.at[pl.ds(lo, CHUNK)], acc.at[pl.ds(0, CHUNK)])

        plsc.subcore_barrier()

        def batch(b, carry):
            pltpu.sync_copy(dst_a.at[pl.ds(e0 + b * BD, BD)], dids)
            for r in range(FB // FL):
                for l in range(FL // 16):
                    locb[r, pl.ds(l * 16, 16)] = trash_vec
                    eidb[r, pl.ds(l * 16, 16)] = zero_vec

            def group(g, cnt):
                ids16 = dids[pl.ds(g * 16, 16)]
                rel = ids16 - lo
                mask = (rel >= 0) & (rel < CHUNK)
                csum = jnp.where(mask, 1, 0)
                for d in (1, 2, 4, 8):
                    sh = csum.at[jnp.maximum(lane - d, 0)].get(
                        mode="promise_in_bounds")
                    csum = csum + jnp.where(lane >= d, sh, 0)
                pos = cnt + csum - 1
                row = lax.shift_right_logical(pos, 6)
                col = lax.bitwise_and(pos, 63)
                plsc.store_scatter(locb, [row, col], rel, mask=mask)
                eidv = jnp.full((16,), e0 + b * BD + g * 16, jnp.int32) + lane
                plsc.store_scatter(eidb, [row, col], eidv, mask=mask)
                return cnt + csum[15]

            cnt = lax.fori_loop(0, BD // 16, group, zero_vec)
            nflush = lax.shift_right_logical(cnt[0] + FL - 1, 6)
            rbufs = (rows0, rows1)
            gsems = (sg0, sg1)
            asems = (sa0, sa1)

            def fire_g(f, s):
                pltpu.async_copy(msg.at[eidb.at[f]], rbufs[s], gsems[s])

            def wait_g(f, s):
                pltpu.make_async_copy(msg.at[eidb.at[f]], rbufs[s],
                                      gsems[s]).wait()

            def fire_a(f, s):
                pltpu.async_copy(rbufs[s], acc.at[locb.at[f]], asems[s],
                                 add=True)

            def wait_a(f, s):
                pltpu.make_async_copy(rbufs[s], acc.at[locb.at[f]],
                                      asems[s]).wait()

            @pl.when(nflush > 0)
            def _():
                fire_g(0, 0)

            def fpair(fh, c2):
                for o in range(2):
                    f = fh * 2 + o
                    s = o

                    @pl.when(f < nflush)
                    def _():
                        wait_g(f, s)

                        @pl.when(f + 1 < nflush)
                        def _():
                            @pl.when(f >= 1)
                            def _():
                                wait_a(f - 1, 1 - s)
                            fire_g(f + 1, 1 - s)

                        fire_a(f, s)
                return c2

            lax.fori_loop(0, FB // FL // 2, fpair, None)
            @pl.when(nflush >= 1)
            def _():
                wait_a(0, 0)

            @pl.when(nflush >= 2)
            def _():
                wait_a(0, 1)
            return carry

        lax.fori_loop(0, EPT // BD, batch, None)
        plsc.subcore_barrier()

        @pl.when(sid == 0)
        def _():
            pltpu.sync_copy(acc.at[pl.ds(0, CHUNK)], out_sv.at[pl.ds(lo, CHUNK)])

        plsc.subcore_barrier()

    def step(k, carry):
        run_chunk(k, acc)
        return carry

    lax.fori_loop(0, N_CHUNKS, step, None)


def _sc_scatter(dst_a, msg, hsv):
    f32 = jnp.float32
    out_type = jax.ShapeDtypeStruct((NPAD, HID), f32)
    scratch = [
        pltpu.VMEM((BD,), jnp.int32),
        pltpu.VMEM((FB // FL, FL), jnp.int32),
        pltpu.VMEM((FB // FL, FL), jnp.int32),
        pltpu.VMEM((FL, HID), f32),
        pltpu.VMEM((FL, HID), f32),
        pltpu.VMEM_SHARED((CHUNK + 8, HID), f32),
    ] + [pltpu.SemaphoreType.DMA] * 5
    return pl.kernel(_scatter_body, out_type=out_type, mesh=_mesh(),
                     compiler_params=pltpu.CompilerParams(needs_layout_passes=False),
                     scratch_types=scratch)(dst_a, msg, hsv)


def kernel(h_s, h_v, edge_index, pos, orientation, W1, b1, W2, b2):
    w1a = W1[:, :SD].T
    w1b = W1[:, SD:2 * SD].T
    w1c = W1[:, 2 * SD:2 * SD + VD].T
    we = w1c[0::2, :]
    wo = w1c[1::2, :]
    wgeo = jnp.concatenate([W1[:, 2 * SD + VD:].T,
                            jnp.zeros((5, HID), jnp.float32)], axis=0)
    b1r = b1.reshape(1, HID)
    w2t = W2.T
    b2r = b2.reshape(1, HID)

    T = _node_table(pos, orientation)

    hvb = h_v.astype(jnp.bfloat16)
    lo16 = lax.bitcast_convert_type(hvb[:, 0::2], jnp.uint16).astype(jnp.uint32)
    hi16 = lax.bitcast_convert_type(hvb[:, 1::2], jnp.uint16).astype(jnp.uint32)
    hv_packed = lax.bitcast_convert_type(lo16 | (hi16 << jnp.uint32(16)),
                                         jnp.float32)
    padA = jnp.zeros((N_NODES, HID - SD - VD // 2 - 4), jnp.float32)
    ta = jnp.concatenate([h_s, hv_packed, T, padA], axis=1)
    padB = jnp.zeros((N_NODES, HID - SD - 4), jnp.float32)
    tb = jnp.concatenate([h_s, T, padB], axis=1)

    src_a = edge_index[0]
    dst_a = edge_index[1]
    x1, x2 = _sc_gather(src_a, dst_a, ta, tb)
    msg = _edge_mlp(x1, x2, w1a, w1b, we, wo, wgeo, b1r, w2t, b2r)
    hsv = jnp.concatenate(
        [jnp.concatenate([h_s, h_v], axis=1),
         jnp.zeros((NPAD - N_NODES, HID), jnp.float32)], axis=0)
    out_sv = _sc_scatter(dst_a, msg, hsv)
    return (out_sv[:N_NODES, :SD], out_sv[:N_NODES, SD:])


# speedup vs baseline: 5.7974x; 1.1373x over previous
import jax
import jax.numpy as jnp
from jax import lax
from jax.experimental import pallas as pl
from jax.experimental.pallas import tpu as pltpu
from jax.experimental.pallas import tpu_sc as plsc

N_NODES = 50000
N_EDGES = 800000
SD = 64
VD = 64
HID = 128

NC = 2
NS = 16
NW = NC * NS

E_B = 2000
N_B = 10000


def _mesh():
    return plsc.VectorSubcoreMesh(core_axis_name="c", subcore_axis_name="s",
                                  num_cores=NC, num_subcores=NS)


def _prep_body(ori_ref, c_ref, s_ref):
    a2 = 2.0 * ori_ref[:, :]
    c_ref[:, :] = jnp.cos(a2)
    s_ref[:, :] = jnp.sin(a2)


def _node_table(pos, orientation):
    ori_lm = orientation.reshape(400, 125)
    c2, s2 = pl.pallas_call(
        _prep_body,
        in_specs=[pl.BlockSpec((400, 125), lambda: (0, 0))],
        out_specs=[pl.BlockSpec((400, 125), lambda: (0, 0))] * 2,
        out_shape=[jax.ShapeDtypeStruct((400, 125), jnp.float32)] * 2,
    )(ori_lm)
    return jnp.concatenate(
        [pos, c2.reshape(N_NODES, 1), s2.reshape(N_NODES, 1)], axis=1)


GU = 128
N_UNITS = N_EDGES // GU
UPW = (N_UNITS + NW - 1) // NW
NBUF = 3
J_TOT = UPW + 2


def _gather_body(src_a, dst_a, ta, tb, o_x1, o_x2, sids, dids, b1, b2,
                 si0, si1, si2, sg0, sg1, sg2, ss0, ss1, ss2):
    wid = lax.axis_index("s") * NC + lax.axis_index("c")
    sem_i = (si0, si1, si2)
    sem_g = (sg0, sg1, sg2)
    sem_s = (ss0, ss1, ss2)

    def valid(k):
        return (k >= 0) & (k < UPW) & (wid + k * NW < N_UNITS)

    def fire_ids(k, p):
        eb = (wid + k * NW) * GU
        pltpu.async_copy(src_a.at[pl.ds(eb, GU)], sids.at[p], sem_i[p])
        pltpu.async_copy(dst_a.at[pl.ds(eb, GU)], dids.at[p], sem_i[p])

    def wait_ids(p):
        pltpu.make_async_copy(src_a.at[pl.ds(0, GU)], sids.at[p], sem_i[p]).wait()
        pltpu.make_async_copy(dst_a.at[pl.ds(0, GU)], dids.at[p], sem_i[p]).wait()

    def fire_gathers(p):
        pltpu.async_copy(ta.at[sids.at[p]], b1.at[p], sem_g[p])
        pltpu.async_copy(tb.at[dids.at[p]], b2.at[p], sem_g[p])

    def wait_gathers(p):
        pltpu.make_async_copy(ta.at[sids.at[p]], b1.at[p], sem_g[p]).wait()
        pltpu.make_async_copy(tb.at[dids.at[p]], b2.at[p], sem_g[p]).wait()

    def fire_stores(k, p):
        eb = (wid + k * NW) * GU
        pltpu.async_copy(b1.at[p], o_x1.at[pl.ds(eb, GU)], ss0 if p == 0 else (ss1 if p == 1 else ss2))
        pltpu.async_copy(b2.at[p], o_x2.at[pl.ds(eb, GU)], sem_s[p])

    def wait_stores(p):
        pltpu.make_async_copy(b1.at[p], o_x1.at[pl.ds(0, GU)], sem_s[p]).wait()
        pltpu.make_async_copy(b2.at[p], o_x2.at[pl.ds(0, GU)], sem_s[p]).wait()

    def tri(it, carry):
        for o in range(3):
            j = it * 3 + o
            @pl.when(valid(j))
            def _():
                fire_ids(j, o)

            p1 = (o - 1) % 3

            @pl.when(valid(j - 4))
            def _():
                wait_stores(p1)

            @pl.when(valid(j - 1))
            def _():
                wait_ids(p1)
                fire_gathers(p1)

            p2 = (o - 2) % 3

            @pl.when(valid(j - 2))
            def _():
                wait_gathers(p2)
                fire_stores(j - 2, p2)
        return carry

    lax.fori_loop(0, J_TOT // 3, tri, None)
    for k_off in range(2):
        k = UPW - 2 + k_off
        p = k % 3

        @pl.when(valid(k))
        def _():
            wait_stores(p)


def _sc_gather(src_a, dst_a, ta, tb):
    f32 = jnp.float32
    out_type = [
        jax.ShapeDtypeStruct((N_EDGES, HID), f32),
        jax.ShapeDtypeStruct((N_EDGES, HID), f32),
    ]
    scratch = [
        pltpu.VMEM((NBUF, GU), jnp.int32),
        pltpu.VMEM((NBUF, GU), jnp.int32),
        pltpu.VMEM((NBUF, GU, HID), f32),
        pltpu.VMEM((NBUF, GU, HID), f32),
    ] + [pltpu.SemaphoreType.DMA] * 9
    return pl.kernel(_gather_body, out_type=out_type, mesh=_mesh(),
                     compiler_params=pltpu.CompilerParams(needs_layout_passes=False),
                     scratch_types=scratch)(src_a, dst_a, ta, tb)


def _mlp_body(x1_ref, x2_ref,
              w1a_ref, w1b_ref, we_ref, wo_ref, wgeo_ref, b1_ref,
              w2t_ref, b2_ref, msg_ref):
    x1 = x1_ref[:, :]
    x2 = x2_ref[:, :]
    tt = jnp.transpose(
        jnp.concatenate([x1[:, 96:104], x2[:, 64:72]], axis=1))
    sx = tt[0:1, :]
    sy = tt[1:2, :]
    c2s = tt[2:3, :]
    s2s = tt[3:4, :]
    dxp = tt[8:9, :]
    dyp = tt[9:10, :]
    c2d = tt[10:11, :]
    s2d = tt[11:12, :]
    dx = sx - dxp
    dy = sy - dyp
    r2 = dx * dx + dy * dy
    nz = r2 > 0.0
    inv = jnp.where(nz, 1.0 / jnp.where(nz, r2, 1.0), 0.0)
    dist_r = jnp.sqrt(r2) + 1e-6
    c2p = jnp.where(nz, (dx * dx - dy * dy) * inv, 1.0)
    s2p = 2.0 * dx * dy * inv
    cg_r = c2p * c2d + s2p * s2d
    sg_r = s2p * c2d - c2p * s2d
    cr_r = c2s * c2d + s2s * s2d
    sr_r = s2s * c2d - c2s * s2d
    five = jnp.transpose(jnp.concatenate(
        [dist_r, cg_r, sg_r, cr_r, sr_r,
         jnp.zeros((3, E_B), jnp.float32)], axis=0))
    dist = five[:, 0:1]
    cg = five[:, 1:2]
    sg = five[:, 2:3]
    cr = five[:, 3:4]
    sr = five[:, 4:5]
    hvp = lax.bitcast_convert_type(x1[:, 64:96], jnp.uint32)
    hv_e = lax.bitcast_convert_type(lax.shift_left(hvp, jnp.uint32(16)), jnp.float32)
    hv_o = lax.bitcast_convert_type(
        lax.bitwise_and(hvp, jnp.uint32(0xFFFF0000)), jnp.float32)
    vrx = cr * hv_e - sr * hv_o
    vry = sr * hv_e + cr * hv_o
    hid = jnp.dot(x1[:, 0:SD], w1a_ref[:, :], preferred_element_type=jnp.float32)
    hid += jnp.dot(x2[:, 0:SD], w1b_ref[:, :], preferred_element_type=jnp.float32)
    hid += jnp.dot(vrx, we_ref[:, :], preferred_element_type=jnp.float32)
    hid += jnp.dot(vry, wo_ref[:, :], preferred_element_type=jnp.float32)
    hid += dist * wgeo_ref[0:1, :] + cg * wgeo_ref[1:2, :] + sg * wgeo_ref[2:3, :]
    hid += b1_ref[:, :]
    hid = hid * (1.0 / (1.0 + jnp.exp(-hid)))
    msg_ref[:, :] = (jnp.dot(hid, w2t_ref[:, :], preferred_element_type=jnp.float32)
                     + b2_ref[:, :])


def _edge_mlp(x1, x2, w1a, w1b, we, wo, wgeo, b1, w2t, b2):
    ew = lambda c: pl.BlockSpec((E_B, c), lambda i: (i, 0))
    full = lambda r, c: pl.BlockSpec((r, c), lambda i: (0, 0))
    return pl.pallas_call(
        _mlp_body,
        grid=(N_EDGES // E_B,),
        in_specs=[
            ew(HID), ew(HID),
            full(SD, HID), full(SD, HID), full(VD // 2, HID), full(VD // 2, HID),
            full(8, HID), full(1, HID), full(HID, HID), full(1, HID),
        ],
        out_specs=ew(HID),
        out_shape=jax.ShapeDtypeStruct((N_EDGES, HID), jnp.float32),
    )(x1, x2, w1a, w1b, we, wo, wgeo, b1, w2t, b2)


NPAD = 50016
CHUNK = 12504
N_CHUNKS = 2
TRASH = CHUNK
EPT = N_EDGES // NS
BD = 2000
FB = 2048
FL = 64


def _scatter_body(dst_a, msg, hsv, out_sv, dids, locb, eidb, rows0, rows1,
                  acc, sem, sg0, sg1, sa0, sa1):
    cid = lax.axis_index("c")
    sid = lax.axis_index("s")
    e0 = sid * EPT
    trash_vec = jnp.full((16,), TRASH, jnp.int32)
    zero_vec = jnp.zeros((16,), jnp.int32)
    lane = lax.iota(jnp.int32, 16)

    def run_chunk(k, acc):
        lo = cid * (CHUNK * N_CHUNKS) + k * CHUNK

        @pl.when(sid == 0)
        def _():
            pltpu.sync_copy(hsv.at[pl.ds(lo, CHUNK)], acc.at[pl.ds(0, CHUNK)])

        plsc.subcore_barrier()

        def batch(b, carry):
            pltpu.sync_copy(dst_a.at[pl.ds(e0 + b * BD, BD)], dids)
            for r in range(FB // FL):
                for l in range(FL // 16):
                    locb[r, pl.ds(l * 16, 16)] = trash_vec
                    eidb[r, pl.ds(l * 16, 16)] = zero_vec

            def group(g, cnt):
                ids16 = dids[pl.ds(g * 16, 16)]
                rel = ids16 - lo
                mask = (rel >= 0) & (rel < CHUNK)
                csum = jnp.where(mask, 1, 0)
                for d in (1, 2, 4, 8):
                    sh = csum.at[jnp.maximum(lane - d, 0)].get(
                        mode="promise_in_bounds")
                    csum = csum + jnp.where(lane >= d, sh, 0)
                pos = cnt + csum - 1
                row = lax.shift_right_logical(pos, 6)
                col = lax.bitwise_and(pos, 63)
                plsc.store_scatter(locb, [row, col], rel, mask=mask)
                eidv = jnp.full((16,), e0 + b * BD + g * 16, jnp.int32) + lane
                plsc.store_scatter(eidb, [row, col], eidv, mask=mask)
                return cnt + csum[15]

            cnt = lax.fori_loop(0, BD // 16, group, zero_vec)
            nflush = lax.shift_right_logical(cnt[0] + FL - 1, 6)
            rbufs = (rows0, rows1)
            gsems = (sg0, sg1)
            asems = (sa0, sa1)

            def fire_g(f, s):
                pltpu.async_copy(msg.at[eidb.at[f]], rbufs[s], gsems[s])

            def wait_g(f, s):
                pltpu.make_async_copy(msg.at[eidb.at[f]], rbufs[s],
                                      gsems[s]).wait()

            def fire_a(f, s):
                pltpu.async_copy(rbufs[s], acc.at[locb.at[f]], asems[s],
                                 add=True)

            def wait_a(f, s):
                pltpu.make_async_copy(rbufs[s], acc.at[locb.at[f]],
                                      asems[s]).wait()

            @pl.when(nflush > 0)
            def _():
                fire_g(0, 0)

            def fpair(fh, c2):
                for o in range(2):
                    f = fh * 2 + o
                    s = o

                    @pl.when(f < nflush)
                    def _():
                        wait_g(f, s)

                        @pl.when(f + 1 < nflush)
                        def _():
                            @pl.when(f >= 1)
                            def _():
                                wait_a(f - 1, 1 - s)
                            fire_g(f + 1, 1 - s)

                        fire_a(f, s)
                return c2

            lax.fori_loop(0, FB // FL // 2, fpair, None)
            @pl.when(nflush >= 1)
            def _():
                wait_a(0, 0)

            @pl.when(nflush >= 2)
            def _():
                wait_a(0, 1)
            return carry

        lax.fori_loop(0, EPT // BD, batch, None)
        plsc.subcore_barrier()

        @pl.when(sid == 0)
        def _():
            pltpu.sync_copy(acc.at[pl.ds(0, CHUNK)], out_sv.at[pl.ds(lo, CHUNK)])

        plsc.subcore_barrier()

    def step(k, carry):
        run_chunk(k, acc)
        return carry

    lax.fori_loop(0, N_CHUNKS, step, None)


def _sc_scatter(dst_a, msg, hsv):
    f32 = jnp.float32
    out_type = jax.ShapeDtypeStruct((NPAD, HID), f32)
    scratch = [
        pltpu.VMEM((BD,), jnp.int32),
        pltpu.VMEM((FB // FL, FL), jnp.int32),
        pltpu.VMEM((FB // FL, FL), jnp.int32),
        pltpu.VMEM((FL, HID), f32),
        pltpu.VMEM((FL, HID), f32),
        pltpu.VMEM_SHARED((CHUNK + 8, HID), f32),
    ] + [pltpu.SemaphoreType.DMA] * 5
    return pl.kernel(_scatter_body, out_type=out_type, mesh=_mesh(),
                     compiler_params=pltpu.CompilerParams(needs_layout_passes=False),
                     scratch_types=scratch)(dst_a, msg, hsv)


def kernel(h_s, h_v, edge_index, pos, orientation, W1, b1, W2, b2):
    w1a = W1[:, :SD].T
    w1b = W1[:, SD:2 * SD].T
    w1c = W1[:, 2 * SD:2 * SD + VD].T
    we = w1c[0::2, :]
    wo = w1c[1::2, :]
    wgeo = jnp.concatenate([W1[:, 2 * SD + VD:].T,
                            jnp.zeros((5, HID), jnp.float32)], axis=0)
    b1r = b1.reshape(1, HID)
    w2t = W2.T
    b2r = b2.reshape(1, HID)

    T = _node_table(pos, orientation)

    hvb = h_v.astype(jnp.bfloat16)
    lo16 = lax.bitcast_convert_type(hvb[:, 0::2], jnp.uint16).astype(jnp.uint32)
    hi16 = lax.bitcast_convert_type(hvb[:, 1::2], jnp.uint16).astype(jnp.uint32)
    hv_packed = lax.bitcast_convert_type(lo16 | (hi16 << jnp.uint32(16)),
                                         jnp.float32)
    padA = jnp.zeros((N_NODES, HID - SD - VD // 2 - 4), jnp.float32)
    ta = jnp.concatenate([h_s, hv_packed, T, padA], axis=1)
    padB = jnp.zeros((N_NODES, HID - SD - 4), jnp.float32)
    tb = jnp.concatenate([h_s, T, padB], axis=1)

    src_a = edge_index[0]
    dst_a = edge_index[1]
    x1, x2 = _sc_gather(src_a, dst_a, ta, tb)
    msg = _edge_mlp(x1, x2, w1a, w1b, we, wo, wgeo, b1r, w2t, b2r)
    hsv = jnp.concatenate(
        [jnp.concatenate([h_s, h_v], axis=1),
         jnp.zeros((NPAD - N_NODES, HID), jnp.float32)], axis=0)
    out_sv = _sc_scatter(dst_a, msg, hsv)
    return (out_sv[:N_NODES, :SD], out_sv[:N_NODES, SD:])


# speedup vs baseline: 7.0359x; 1.2136x over previous
import jax
import jax.numpy as jnp
from jax import lax
from jax.experimental import pallas as pl
from jax.experimental.pallas import tpu as pltpu
from jax.experimental.pallas import tpu_sc as plsc

N_NODES = 50000
N_EDGES = 800000
SD = 64
VD = 64
HID = 128

NC = 2
NS = 16
NW = NC * NS

E_B = 2000
N_B = 10000


def _mesh():
    return plsc.VectorSubcoreMesh(core_axis_name="c", subcore_axis_name="s",
                                  num_cores=NC, num_subcores=NS)


def _prep_body(ori_ref, c_ref, s_ref):
    a2 = 2.0 * ori_ref[:, :]
    c_ref[:, :] = jnp.cos(a2)
    s_ref[:, :] = jnp.sin(a2)


def _node_table(pos, orientation):
    ori_lm = orientation.reshape(400, 125)
    c2, s2 = pl.pallas_call(
        _prep_body,
        in_specs=[pl.BlockSpec((400, 125), lambda: (0, 0))],
        out_specs=[pl.BlockSpec((400, 125), lambda: (0, 0))] * 2,
        out_shape=[jax.ShapeDtypeStruct((400, 125), jnp.float32)] * 2,
    )(ori_lm)
    return jnp.concatenate(
        [pos, c2.reshape(N_NODES, 1), s2.reshape(N_NODES, 1)], axis=1)


GU = 128
N_UNITS = N_EDGES // GU
UPW = (N_UNITS + NW - 1) // NW
NBUF = 3
J_TOT = UPW + 2


def _gather_body(src_a, dst_a, ta, tb, o_x1, o_x2, sids, dids, b1, b2,
                 si0, si1, si2, sg0, sg1, sg2, ss0, ss1, ss2):
    wid = lax.axis_index("s") * NC + lax.axis_index("c")
    sem_i = (si0, si1, si2)
    sem_g = (sg0, sg1, sg2)
    sem_s = (ss0, ss1, ss2)

    def valid(k):
        return (k >= 0) & (k < UPW) & (wid + k * NW < N_UNITS)

    def fire_ids(k, p):
        eb = (wid + k * NW) * GU
        pltpu.async_copy(src_a.at[pl.ds(eb, GU)], sids.at[p], sem_i[p])
        pltpu.async_copy(dst_a.at[pl.ds(eb, GU)], dids.at[p], sem_i[p])

    def wait_ids(p):
        pltpu.make_async_copy(src_a.at[pl.ds(0, GU)], sids.at[p], sem_i[p]).wait()
        pltpu.make_async_copy(dst_a.at[pl.ds(0, GU)], dids.at[p], sem_i[p]).wait()

    def fire_gathers(p):
        pltpu.async_copy(ta.at[sids.at[p]], b1.at[p], sem_g[p])
        pltpu.async_copy(tb.at[dids.at[p]], b2.at[p], sem_g[p])

    def wait_gathers(p):
        pltpu.make_async_copy(ta.at[sids.at[p]], b1.at[p], sem_g[p]).wait()
        pltpu.make_async_copy(tb.at[dids.at[p]], b2.at[p], sem_g[p]).wait()

    def fire_stores(k, p):
        eb = (wid + k * NW) * GU
        pltpu.async_copy(b1.at[p], o_x1.at[pl.ds(eb, GU)], ss0 if p == 0 else (ss1 if p == 1 else ss2))
        pltpu.async_copy(b2.at[p], o_x2.at[pl.ds(eb, GU)], sem_s[p])

    def wait_stores(p):
        pltpu.make_async_copy(b1.at[p], o_x1.at[pl.ds(0, GU)], sem_s[p]).wait()
        pltpu.make_async_copy(b2.at[p], o_x2.at[pl.ds(0, GU)], sem_s[p]).wait()

    def tri(it, carry):
        for o in range(3):
            j = it * 3 + o
            @pl.when(valid(j))
            def _():
                fire_ids(j, o)

            p1 = (o - 1) % 3

            @pl.when(valid(j - 4))
            def _():
                wait_stores(p1)

            @pl.when(valid(j - 1))
            def _():
                wait_ids(p1)
                fire_gathers(p1)

            p2 = (o - 2) % 3

            @pl.when(valid(j - 2))
            def _():
                wait_gathers(p2)
                fire_stores(j - 2, p2)
        return carry

    lax.fori_loop(0, J_TOT // 3, tri, None)
    for k_off in range(2):
        k = UPW - 2 + k_off
        p = k % 3

        @pl.when(valid(k))
        def _():
            wait_stores(p)


def _sc_gather(src_a, dst_a, ta, tb):
    f32 = jnp.float32
    out_type = [
        jax.ShapeDtypeStruct((N_EDGES, HID), f32),
        jax.ShapeDtypeStruct((N_EDGES, HID), f32),
    ]
    scratch = [
        pltpu.VMEM((NBUF, GU), jnp.int32),
        pltpu.VMEM((NBUF, GU), jnp.int32),
        pltpu.VMEM((NBUF, GU, HID), f32),
        pltpu.VMEM((NBUF, GU, HID), f32),
    ] + [pltpu.SemaphoreType.DMA] * 9
    return pl.kernel(_gather_body, out_type=out_type, mesh=_mesh(),
                     compiler_params=pltpu.CompilerParams(needs_layout_passes=False),
                     scratch_types=scratch)(src_a, dst_a, ta, tb)


def _mlp_body(x1_ref, x2_ref,
              w1a_ref, w1b_ref, we_ref, wo_ref, wgeo_ref, b1_ref,
              w2t_ref, b2_ref, msg_ref):
    x1 = x1_ref[:, :]
    x2 = x2_ref[:, :]
    tt = jnp.transpose(
        jnp.concatenate([x1[:, 96:104], x2[:, 64:72]], axis=1))
    sx = tt[0:1, :]
    sy = tt[1:2, :]
    c2s = tt[2:3, :]
    s2s = tt[3:4, :]
    dxp = tt[8:9, :]
    dyp = tt[9:10, :]
    c2d = tt[10:11, :]
    s2d = tt[11:12, :]
    dx = sx - dxp
    dy = sy - dyp
    r2 = dx * dx + dy * dy
    nz = r2 > 0.0
    inv = jnp.where(nz, 1.0 / jnp.where(nz, r2, 1.0), 0.0)
    dist_r = jnp.sqrt(r2) + 1e-6
    c2p = jnp.where(nz, (dx * dx - dy * dy) * inv, 1.0)
    s2p = 2.0 * dx * dy * inv
    cg_r = c2p * c2d + s2p * s2d
    sg_r = s2p * c2d - c2p * s2d
    cr_r = c2s * c2d + s2s * s2d
    sr_r = s2s * c2d - c2s * s2d
    five = jnp.transpose(jnp.concatenate(
        [dist_r, cg_r, sg_r, cr_r, sr_r,
         jnp.zeros((3, E_B), jnp.float32)], axis=0))
    dist = five[:, 0:1]
    cg = five[:, 1:2]
    sg = five[:, 2:3]
    cr = five[:, 3:4]
    sr = five[:, 4:5]
    hvp = lax.bitcast_convert_type(x1[:, 64:96], jnp.uint32)
    hv_e = lax.bitcast_convert_type(lax.shift_left(hvp, jnp.uint32(16)), jnp.float32)
    hv_o = lax.bitcast_convert_type(
        lax.bitwise_and(hvp, jnp.uint32(0xFFFF0000)), jnp.float32)
    vrx = cr * hv_e - sr * hv_o
    vry = sr * hv_e + cr * hv_o
    hid = jnp.dot(x1[:, 0:SD], w1a_ref[:, :], preferred_element_type=jnp.float32)
    hid += jnp.dot(x2[:, 0:SD], w1b_ref[:, :], preferred_element_type=jnp.float32)
    hid += jnp.dot(vrx, we_ref[:, :], preferred_element_type=jnp.float32)
    hid += jnp.dot(vry, wo_ref[:, :], preferred_element_type=jnp.float32)
    hid += dist * wgeo_ref[0:1, :] + cg * wgeo_ref[1:2, :] + sg * wgeo_ref[2:3, :]
    hid += b1_ref[:, :]
    hid = hid * (1.0 / (1.0 + jnp.exp(-hid)))
    msg_ref[:, :] = (jnp.dot(hid, w2t_ref[:, :], preferred_element_type=jnp.float32)
                     + b2_ref[:, :])


def _edge_mlp(x1, x2, w1a, w1b, we, wo, wgeo, b1, w2t, b2):
    ew = lambda c: pl.BlockSpec((E_B, c), lambda i: (i, 0))
    full = lambda r, c: pl.BlockSpec((r, c), lambda i: (0, 0))
    return pl.pallas_call(
        _mlp_body,
        grid=(N_EDGES // E_B,),
        in_specs=[
            ew(HID), ew(HID),
            full(SD, HID), full(SD, HID), full(VD // 2, HID), full(VD // 2, HID),
            full(8, HID), full(1, HID), full(HID, HID), full(1, HID),
        ],
        out_specs=ew(HID),
        out_shape=jax.ShapeDtypeStruct((N_EDGES, HID), jnp.float32),
    )(x1, x2, w1a, w1b, we, wo, wgeo, b1, w2t, b2)


NPAD = 50016
CHUNK = 12504
N_CHUNKS = 2
TRASH = CHUNK
EPT = N_EDGES // NS
BD = 2000
FB = 2048
FL = 32


def _scatter_body(dst_a, msg, hsv, out_sv, dids, locb, eidb, rows0, rows1,
                  rows2, acc, sem, sg0, sg1, sg2, sa0, sa1, sa2):
    cid = lax.axis_index("c")
    sid = lax.axis_index("s")
    e0 = sid * EPT
    trash_vec = jnp.full((16,), TRASH, jnp.int32)
    zero_vec = jnp.zeros((16,), jnp.int32)
    lane = lax.iota(jnp.int32, 16)

    def run_chunk(k, acc):
        lo = cid * (CHUNK * N_CHUNKS) + k * CHUNK

        @pl.when(sid == 0)
        def _():
            pltpu.sync_copy(hsv.at[pl.ds(lo, CHUNK)], acc.at[pl.ds(0, CHUNK)])

        plsc.subcore_barrier()

        def batch(b, carry):
            pltpu.sync_copy(dst_a.at[pl.ds(e0 + b * BD, BD)], dids)
            for r in range(FB // FL):
                for l in range(FL // 16):
                    locb[r, pl.ds(l * 16, 16)] = trash_vec
                    eidb[r, pl.ds(l * 16, 16)] = zero_vec

            def group(g, cnt):
                ids16 = dids[pl.ds(g * 16, 16)]
                rel = ids16 - lo
                mask = (rel >= 0) & (rel < CHUNK)
                csum = jnp.where(mask, 1, 0)
                for d in (1, 2, 4, 8):
                    sh = csum.at[jnp.maximum(lane - d, 0)].get(
                        mode="promise_in_bounds")
                    csum = csum + jnp.where(lane >= d, sh, 0)
                pos = cnt + csum - 1
                row = lax.shift_right_logical(pos, 5)
                col = lax.bitwise_and(pos, 31)
                plsc.store_scatter(locb, [row, col], rel, mask=mask)
                eidv = jnp.full((16,), e0 + b * BD + g * 16, jnp.int32) + lane
                plsc.store_scatter(eidb, [row, col], eidv, mask=mask)
                return cnt + csum[15]

            cnt = lax.fori_loop(0, BD // 16, group, zero_vec)
            nflush = lax.shift_right_logical(cnt[0] + FL - 1, 5)
            rbufs = (rows0, rows1, rows2)
            gsems = (sg0, sg1, sg2)
            asems = (sa0, sa1, sa2)

            def fire_g(f, s):
                pltpu.async_copy(msg.at[eidb.at[f]], rbufs[s], gsems[s])

            def wait_g(f, s):
                pltpu.make_async_copy(msg.at[eidb.at[f]], rbufs[s],
                                      gsems[s]).wait()

            def fire_a(f, s):
                pltpu.async_copy(rbufs[s], acc.at[locb.at[f]], asems[s],
                                 add=True)

            def wait_a(f, s):
                pltpu.make_async_copy(rbufs[s], acc.at[locb.at[f]],
                                      asems[s]).wait()

            @pl.when(nflush > 0)
            def _():
                fire_g(0, 0)

            @pl.when(nflush > 1)
            def _():
                fire_g(1, 1)

            def ftri(fh, c2):
                for o in range(3):
                    f = fh * 3 + o
                    s = o

                    @pl.when(f < nflush)
                    def _():
                        wait_g(f, s)

                        @pl.when(f + 2 < nflush)
                        def _():
                            @pl.when(f >= 1)
                            def _():
                                wait_a(f - 1, (s + 2) % 3)
                            fire_g(f + 2, (s + 2) % 3)

                        fire_a(f, s)
                return c2

            lax.fori_loop(0, (FB // FL + 2) // 3, ftri, None)
            @pl.when(nflush >= 1)
            def _():
                wait_a(0, 0)

            @pl.when(nflush >= 2)
            def _():
                wait_a(0, 1)

            @pl.when(nflush >= 3)
            def _():
                wait_a(0, 2)
            return carry

        lax.fori_loop(0, EPT // BD, batch, None)
        plsc.subcore_barrier()

        @pl.when(sid == 0)
        def _():
            pltpu.sync_copy(acc.at[pl.ds(0, CHUNK)], out_sv.at[pl.ds(lo, CHUNK)])

        plsc.subcore_barrier()

    def step(k, carry):
        run_chunk(k, acc)
        return carry

    lax.fori_loop(0, N_CHUNKS, step, None)


def _sc_scatter(dst_a, msg, hsv):
    f32 = jnp.float32
    out_type = jax.ShapeDtypeStruct((NPAD, HID), f32)
    scratch = [
        pltpu.VMEM((BD,), jnp.int32),
        pltpu.VMEM((FB // FL, FL), jnp.int32),
        pltpu.VMEM((FB // FL, FL), jnp.int32),
        pltpu.VMEM((FL, HID), f32),
        pltpu.VMEM((FL, HID), f32),
        pltpu.VMEM((FL, HID), f32),
        pltpu.VMEM_SHARED((CHUNK + 8, HID), f32),
    ] + [pltpu.SemaphoreType.DMA] * 7
    return pl.kernel(_scatter_body, out_type=out_type, mesh=_mesh(),
                     compiler_params=pltpu.CompilerParams(needs_layout_passes=False),
                     scratch_types=scratch)(dst_a, msg, hsv)


def kernel(h_s, h_v, edge_index, pos, orientation, W1, b1, W2, b2):
    w1a = W1[:, :SD].T
    w1b = W1[:, SD:2 * SD].T
    w1c = W1[:, 2 * SD:2 * SD + VD].T
    we = w1c[0::2, :]
    wo = w1c[1::2, :]
    wgeo = jnp.concatenate([W1[:, 2 * SD + VD:].T,
                            jnp.zeros((5, HID), jnp.float32)], axis=0)
    b1r = b1.reshape(1, HID)
    w2t = W2.T
    b2r = b2.reshape(1, HID)

    T = _node_table(pos, orientation)

    hvb = h_v.astype(jnp.bfloat16)
    lo16 = lax.bitcast_convert_type(hvb[:, 0::2], jnp.uint16).astype(jnp.uint32)
    hi16 = lax.bitcast_convert_type(hvb[:, 1::2], jnp.uint16).astype(jnp.uint32)
    hv_packed = lax.bitcast_convert_type(lo16 | (hi16 << jnp.uint32(16)),
                                         jnp.float32)
    padA = jnp.zeros((N_NODES, HID - SD - VD // 2 - 4), jnp.float32)
    ta = jnp.concatenate([h_s, hv_packed, T, padA], axis=1)
    padB = jnp.zeros((N_NODES, HID - SD - 4), jnp.float32)
    tb = jnp.concatenate([h_s, T, padB], axis=1)

    src_a = edge_index[0]
    dst_a = edge_index[1]
    x1, x2 = _sc_gather(src_a, dst_a, ta, tb)
    msg = _edge_mlp(x1, x2, w1a, w1b, we, wo, wgeo, b1r, w2t, b2r)
    hsv = jnp.concatenate(
        [jnp.concatenate([h_s, h_v], axis=1),
         jnp.zeros((NPAD - N_NODES, HID), jnp.float32)], axis=0)
    out_sv = _sc_scatter(dst_a, msg, hsv)
    return (out_sv[:N_NODES, :SD], out_sv[:N_NODES, SD:])
